# baseline (speedup 1.0000x reference)
import dataclasses
import functools

import jax
import jax.numpy as jnp
from jax import lax
from jax.experimental import pallas as pl
from jax.experimental.pallas import tpu as pltpu
from jax.experimental.pallas import tpu_sc as plsc

N = 10000
E = 320000
D = 128
NSC = 2
NSUB = 16
NW = NSC * NSUB
CH = 80
K = 16
NG = 8
NCH = NG * K
EPW = NCH * CH
EPAD = NW * EPW
NP = 10240
STRIPE = NP // NSUB


def _sc_aggregate(feature, ei4, zrows):
    mesh = plsc.VectorSubcoreMesh(core_axis_name="c", subcore_axis_name="s")
    cp = pltpu.CompilerParams()
    if "needs_layout_passes" in pltpu.CompilerParams.__dataclass_fields__:
        cp = dataclasses.replace(cp, needs_layout_passes=False)

    @functools.partial(
        pl.kernel,
        mesh=mesh,
        compiler_params=cp,
        out_type=(jax.ShapeDtypeStruct((NSC, NP, D), jnp.float32),
                  jax.ShapeDtypeStruct((NW, NP), jnp.float32)),
        scratch_types=[
            pltpu.VMEM_SHARED((NP, D), jnp.float32),
            pltpu.VMEM((1, CH), jnp.int32),
            pltpu.VMEM((1, CH), jnp.int32),
            pltpu.VMEM((CH, D), jnp.float32),
            pltpu.VMEM((NP,), jnp.float32),
        ],
    )
    def k(f_hbm, ei_hbm, z_hbm, sums_hbm, cnt_hbm, acc_sh, src_v, dst_v,
          rows_v, hist_v):
        cid = lax.axis_index("c")
        sid = lax.axis_index("s")
        wid = cid * NSUB + sid
        base = wid * EPW

        pltpu.sync_copy(z_hbm, acc_sh.at[pl.ds(sid * STRIPE, STRIPE)])

        @pl.loop(0, NP, step=16)
        def _(i):
            hist_v[pl.ds(i, 16)] = jnp.zeros((16,), jnp.float32)

        plsc.subcore_barrier()
        ones16 = jnp.ones((16,), jnp.float32)

        @pl.loop(0, NCH)
        def _(i):
            off = base + i * CH
            pltpu.sync_copy(ei_hbm.at[pl.ds(off, CH)], src_v.at[0])
            pltpu.sync_copy(ei_hbm.at[pl.ds(EPAD + off, CH)], dst_v.at[0])
            pltpu.sync_copy(f_hbm.at[src_v.at[0]], rows_v)
            pltpu.sync_copy(rows_v, acc_sh.at[dst_v.at[0]], add=True)
            for j in range(CH // 16):
                iv = dst_v[0, pl.ds(j * 16, 16)]
                plsc.addupdate_scatter(hist_v, [iv], ones16)

        pltpu.sync_copy(hist_v, cnt_hbm.at[wid])
        plsc.subcore_barrier()
        pltpu.sync_copy(acc_sh.at[pl.ds(sid * STRIPE, STRIPE)],
                        sums_hbm.at[cid, pl.ds(sid * STRIPE, STRIPE)])

    return k(feature, ei4, zrows)


def _tc_epilogue(acc, cnt, feature, W, b2):
    def body(acc_ref, c_ref, f_ref, w_ref, b_ref, o_ref):
        sums = acc_ref[0, :N, :] + acc_ref[1, :N, :]
        agg = sums / jnp.maximum(c_ref[...], 1.0)
        h = (jnp.dot(agg, w_ref[:D, :], preferred_element_type=jnp.float32)
             + jnp.dot(f_ref[...], w_ref[D:, :], preferred_element_type=jnp.float32)
             + b_ref[...])
        nrm2 = jnp.sum(h * h, axis=1, keepdims=True)
        o_ref[...] = h * lax.rsqrt(jnp.maximum(nrm2, 1e-24))

    return pl.pallas_call(
        body,
        out_shape=jax.ShapeDtypeStruct((N, D), jnp.float32),
    )(acc, cnt, feature, W, b2)


def kernel(feature, edge_index, W, b):
    npad = EPAD - E
    pad_src = jnp.zeros((1, npad), jnp.int32)
    pad_dst = (N + jnp.arange(npad, dtype=jnp.int32) % (NP - N))[None, :]
    ei4 = jnp.concatenate(
        [edge_index, jnp.concatenate([pad_src, pad_dst], axis=0)],
        axis=1).reshape(-1)
    zrows = jnp.zeros((STRIPE, D), jnp.float32)
    acc, cparts = _sc_aggregate(feature, ei4, zrows)
    cnt = cparts.sum(axis=0)[:N, None]
    return _tc_epilogue(acc, cnt, feature, W, b.reshape(1, D))


# speedup vs baseline: 2.0734x; 2.0734x over previous
import dataclasses
import functools

import jax
import jax.numpy as jnp
from jax import lax
from jax.experimental import pallas as pl
from jax.experimental.pallas import tpu as pltpu
from jax.experimental.pallas import tpu_sc as plsc

N = 10000
E = 320000
D = 128
NSC = 2
NSUB = 16
NW = NSC * NSUB
CH = 80
NCH = 125
EPW = NCH * CH
NP = 10240
STRIPE = NP // NSUB


def _sc_aggregate(feature, ei4, zrows):
    mesh = plsc.VectorSubcoreMesh(core_axis_name="c", subcore_axis_name="s")
    cp = pltpu.CompilerParams()
    if "needs_layout_passes" in pltpu.CompilerParams.__dataclass_fields__:
        cp = dataclasses.replace(cp, needs_layout_passes=False)

    @functools.partial(
        pl.kernel,
        mesh=mesh,
        compiler_params=cp,
        out_type=(jax.ShapeDtypeStruct((NSC, NP, D), jnp.float32),
                  jax.ShapeDtypeStruct((NW, NP), jnp.float32)),
        scratch_types=[
            pltpu.VMEM_SHARED((NP, D), jnp.float32),
            pltpu.VMEM((1, CH), jnp.int32),
            pltpu.VMEM((1, CH), jnp.int32),
            pltpu.VMEM((CH, D), jnp.float32),
            pltpu.VMEM((NP,), jnp.float32),
        ],
    )
    def k(f_hbm, ei_hbm, z_hbm, sums_hbm, cnt_hbm, acc_sh, src_v, dst_v,
          rows_v, hist_v):
        cid = lax.axis_index("c")
        sid = lax.axis_index("s")
        wid = cid * NSUB + sid
        base = wid * EPW

        pltpu.sync_copy(z_hbm, acc_sh.at[pl.ds(sid * STRIPE, STRIPE)])

        @pl.loop(0, NP, step=16)
        def _(i):
            hist_v[pl.ds(i, 16)] = jnp.zeros((16,), jnp.float32)

        plsc.subcore_barrier()
        ones16 = jnp.ones((16,), jnp.float32)

        @pl.loop(0, NCH)
        def _(i):
            off = base + i * CH
            pltpu.sync_copy(ei_hbm.at[pl.ds(off, CH)], src_v.at[0])
            pltpu.sync_copy(ei_hbm.at[pl.ds(E + off, CH)], dst_v.at[0])
            pltpu.sync_copy(f_hbm.at[src_v.at[0]], rows_v)
            pltpu.sync_copy(rows_v, acc_sh.at[dst_v.at[0]], add=True)
            for j in range(CH // 16):
                iv = dst_v[0, pl.ds(j * 16, 16)]
                plsc.addupdate_scatter(hist_v, [iv], ones16)

        pltpu.sync_copy(hist_v, cnt_hbm.at[wid])
        plsc.subcore_barrier()
        pltpu.sync_copy(acc_sh.at[pl.ds(sid * STRIPE, STRIPE)],
                        sums_hbm.at[cid, pl.ds(sid * STRIPE, STRIPE)])

    return k(feature, ei4, zrows)


def _tc_epilogue(acc, cnt, feature, W, b2):
    def body(acc_ref, c_ref, f_ref, w_ref, b_ref, o_ref):
        sums = acc_ref[0, :N, :] + acc_ref[1, :N, :]
        agg = sums / jnp.maximum(c_ref[...], 1.0)
        h = (jnp.dot(agg, w_ref[:D, :], preferred_element_type=jnp.float32)
             + jnp.dot(f_ref[...], w_ref[D:, :], preferred_element_type=jnp.float32)
             + b_ref[...])
        nrm2 = jnp.sum(h * h, axis=1, keepdims=True)
        o_ref[...] = h * lax.rsqrt(jnp.maximum(nrm2, 1e-24))

    return pl.pallas_call(
        body,
        out_shape=jax.ShapeDtypeStruct((N, D), jnp.float32),
    )(acc, cnt, feature, W, b2)


def kernel(feature, edge_index, W, b):
    zrows = jnp.zeros((STRIPE, D), jnp.float32)
    acc, cparts = _sc_aggregate(feature, edge_index.reshape(-1), zrows)
    cnt = cparts.sum(axis=0)[:N, None]
    return _tc_epilogue(acc, cnt, feature, W, b.reshape(1, D))


# speedup vs baseline: 3.3886x; 1.6343x over previous
import dataclasses
import functools

import jax
import jax.numpy as jnp
from jax import lax
from jax.experimental import pallas as pl
from jax.experimental.pallas import tpu as pltpu
from jax.experimental.pallas import tpu_sc as plsc

N = 10000
E = 320000
D = 128
NSC = 2
NSUB = 16
NW = NSC * NSUB
CH = 80
K = 5
NG = 25
NCH = NG * K
EPW = NCH * CH
NP = 10240
STRIPE = NP // NSUB


def _sc_aggregate(feature, ei4, zrows):
    mesh = plsc.VectorSubcoreMesh(core_axis_name="c", subcore_axis_name="s")
    cp = pltpu.CompilerParams()
    if "needs_layout_passes" in pltpu.CompilerParams.__dataclass_fields__:
        cp = dataclasses.replace(cp, needs_layout_passes=False)

    @functools.partial(
        pl.kernel,
        mesh=mesh,
        compiler_params=cp,
        out_type=(jax.ShapeDtypeStruct((NSC, NP, D), jnp.float32),
                  jax.ShapeDtypeStruct((NW, NP), jnp.float32)),
        scratch_types=[
            pltpu.VMEM_SHARED((NP, D), jnp.float32),
            pltpu.VMEM((K, CH), jnp.int32),
            pltpu.VMEM((K, CH), jnp.int32),
            pltpu.VMEM((2, CH, D), jnp.float32),
            pltpu.VMEM((NP,), jnp.float32),
            pltpu.SemaphoreType.DMA,
            pltpu.SemaphoreType.DMA,
        ],
    )
    def k(f_hbm, ei_hbm, z_hbm, sums_hbm, cnt_hbm, acc_sh, src_v, dst_v,
          rows_v, hist_v, sem0, sem1):
        cid = lax.axis_index("c")
        sid = lax.axis_index("s")
        wid = cid * NSUB + sid
        base = wid * EPW

        pltpu.sync_copy(z_hbm, acc_sh.at[pl.ds(sid * STRIPE, STRIPE)])

        @pl.loop(0, NP, step=16)
        def _(i):
            hist_v[pl.ds(i, 16)] = jnp.zeros((16,), jnp.float32)

        plsc.subcore_barrier()
        ones16 = jnp.ones((16,), jnp.float32)
        sems = (sem0, sem1)

        @pl.loop(0, NG)
        def _(g):
            pltpu.sync_copy(ei_hbm.at[0, wid, g], src_v)
            pltpu.sync_copy(ei_hbm.at[1, wid, g], dst_v)
            pltpu.async_copy(f_hbm.at[src_v.at[0]], rows_v.at[0], sem0)
            for c in range(K):
                b = c % 2
                if c + 1 < K:
                    pltpu.async_copy(f_hbm.at[src_v.at[c + 1]],
                                     rows_v.at[1 - b], sems[1 - b])
                for j in range(CH // 16):
                    iv = dst_v[c, pl.ds(j * 16, 16)]
                    plsc.addupdate_scatter(hist_v, [iv], ones16)
                pltpu.make_async_copy(f_hbm.at[src_v.at[c]],
                                      rows_v.at[b], sems[b]).wait()
                pltpu.sync_copy(rows_v.at[b], acc_sh.at[dst_v.at[c]],
                                add=True)

        pltpu.sync_copy(hist_v, cnt_hbm.at[wid])
        plsc.subcore_barrier()
        pltpu.sync_copy(acc_sh.at[pl.ds(sid * STRIPE, STRIPE)],
                        sums_hbm.at[cid, pl.ds(sid * STRIPE, STRIPE)])

    return k(feature, ei4, zrows)


def _tc_epilogue(acc, cnt, feature, W, b2):
    def body(acc_ref, c_ref, f_ref, w_ref, b_ref, o_ref):
        sums = acc_ref[0, :N, :] + acc_ref[1, :N, :]
        agg = sums / jnp.maximum(c_ref[...], 1.0)
        h = (jnp.dot(agg, w_ref[:D, :], preferred_element_type=jnp.float32)
             + jnp.dot(f_ref[...], w_ref[D:, :], preferred_element_type=jnp.float32)
             + b_ref[...])
        nrm2 = jnp.sum(h * h, axis=1, keepdims=True)
        o_ref[...] = h * lax.rsqrt(jnp.maximum(nrm2, 1e-24))

    return pl.pallas_call(
        body,
        out_shape=jax.ShapeDtypeStruct((N, D), jnp.float32),
    )(acc, cnt, feature, W, b2)


def kernel(feature, edge_index, W, b):
    zrows = jnp.zeros((STRIPE, D), jnp.float32)
    acc, cparts = _sc_aggregate(
        feature, edge_index.reshape(2, NW, NG, K, CH), zrows)
    cnt = cparts.sum(axis=0)[:N, None]
    return _tc_epilogue(acc, cnt, feature, W, b.reshape(1, D))


# speedup vs baseline: 3.3923x; 1.0011x over previous
import dataclasses
import functools

import jax
import jax.numpy as jnp
from jax import lax
from jax.experimental import pallas as pl
from jax.experimental.pallas import tpu as pltpu
from jax.experimental.pallas import tpu_sc as plsc

N = 10000
E = 320000
D = 128
NSC = 2
NSUB = 16
NW = NSC * NSUB
CH = 80
K = 5
NG = 25
NCH = NG * K
EPW = NCH * CH
NP = 10240
STRIPE = NP // NSUB


def _sc_aggregate(feature, ei4, zrows):
    mesh = plsc.VectorSubcoreMesh(core_axis_name="c", subcore_axis_name="s")
    cp = pltpu.CompilerParams()
    if "needs_layout_passes" in pltpu.CompilerParams.__dataclass_fields__:
        cp = dataclasses.replace(cp, needs_layout_passes=False)

    @functools.partial(
        pl.kernel,
        mesh=mesh,
        compiler_params=cp,
        out_type=(jax.ShapeDtypeStruct((NSC, NP, D), jnp.float32),
                  jax.ShapeDtypeStruct((NW, NP), jnp.float32)),
        scratch_types=[
            pltpu.VMEM_SHARED((NP, D), jnp.float32),
            pltpu.VMEM((K, CH), jnp.int32),
            pltpu.VMEM((K, CH), jnp.int32),
            pltpu.VMEM((2, CH, D), jnp.float32),
            pltpu.VMEM((NP,), jnp.float32),
            pltpu.SemaphoreType.DMA,
            pltpu.SemaphoreType.DMA,
            pltpu.SemaphoreType.DMA,
            pltpu.SemaphoreType.DMA,
        ],
    )
    def k(f_hbm, ei_hbm, z_hbm, sums_hbm, cnt_hbm, acc_sh, src_v, dst_v,
          rows_v, hist_v, sem0, sem1, ssem0, ssem1):
        cid = lax.axis_index("c")
        sid = lax.axis_index("s")
        wid = cid * NSUB + sid
        base = wid * EPW

        pltpu.sync_copy(z_hbm, acc_sh.at[pl.ds(sid * STRIPE, STRIPE)])

        @pl.loop(0, NP, step=16)
        def _(i):
            hist_v[pl.ds(i, 16)] = jnp.zeros((16,), jnp.float32)

        plsc.subcore_barrier()
        ones16 = jnp.ones((16,), jnp.float32)
        sems = (sem0, sem1)
        ssems = (ssem0, ssem1)

        @pl.loop(0, NG)
        def _(g):
            pltpu.sync_copy(ei_hbm.at[0, wid, g], src_v)
            pltpu.sync_copy(ei_hbm.at[1, wid, g], dst_v)
            pltpu.async_copy(f_hbm.at[src_v.at[0]], rows_v.at[0], sem0)
            for c in range(K):
                b = c % 2
                if c + 1 < K:
                    if c >= 1:
                        pltpu.make_async_copy(
                            rows_v.at[1 - b], acc_sh.at[dst_v.at[c - 1]],
                            ssems[1 - b]).wait()
                    pltpu.async_copy(f_hbm.at[src_v.at[c + 1]],
                                     rows_v.at[1 - b], sems[1 - b])
                for j in range(CH // 16):
                    iv = dst_v[c, pl.ds(j * 16, 16)]
                    plsc.addupdate_scatter(hist_v, [iv], ones16)
                pltpu.make_async_copy(f_hbm.at[src_v.at[c]],
                                      rows_v.at[b], sems[b]).wait()
                pltpu.async_copy(rows_v.at[b], acc_sh.at[dst_v.at[c]],
                                 ssems[b], add=True)
            pltpu.make_async_copy(rows_v.at[(K - 2) % 2],
                                  acc_sh.at[dst_v.at[K - 2]],
                                  ssems[(K - 2) % 2]).wait()
            pltpu.make_async_copy(rows_v.at[(K - 1) % 2],
                                  acc_sh.at[dst_v.at[K - 1]],
                                  ssems[(K - 1) % 2]).wait()

        pltpu.sync_copy(hist_v, cnt_hbm.at[wid])
        plsc.subcore_barrier()
        pltpu.sync_copy(acc_sh.at[pl.ds(sid * STRIPE, STRIPE)],
                        sums_hbm.at[cid, pl.ds(sid * STRIPE, STRIPE)])

    return k(feature, ei4, zrows)


def _tc_epilogue(acc, cnt, feature, W, b2):
    def body(acc_ref, c_ref, f_ref, w_ref, b_ref, o_ref):
        sums = acc_ref[0, :N, :] + acc_ref[1, :N, :]
        agg = sums / jnp.maximum(c_ref[...], 1.0)
        h = (jnp.dot(agg, w_ref[:D, :], preferred_element_type=jnp.float32)
             + jnp.dot(f_ref[...], w_ref[D:, :], preferred_element_type=jnp.float32)
             + b_ref[...])
        nrm2 = jnp.sum(h * h, axis=1, keepdims=True)
        o_ref[...] = h * lax.rsqrt(jnp.maximum(nrm2, 1e-24))

    return pl.pallas_call(
        body,
        out_shape=jax.ShapeDtypeStruct((N, D), jnp.float32),
    )(acc, cnt, feature, W, b2)


def kernel(feature, edge_index, W, b):
    zrows = jnp.zeros((STRIPE, D), jnp.float32)
    acc, cparts = _sc_aggregate(
        feature, edge_index.reshape(2, NW, NG, K, CH), zrows)
    cnt = cparts.sum(axis=0)[:N, None]
    return _tc_epilogue(acc, cnt, feature, W, b.reshape(1, D))


# speedup vs baseline: 3.6867x; 1.0868x over previous
import dataclasses
import functools

import jax
import jax.numpy as jnp
from jax import lax
from jax.experimental import pallas as pl
from jax.experimental.pallas import tpu as pltpu
from jax.experimental.pallas import tpu_sc as plsc

N = 10000
E = 320000
D = 128
NSC = 2
NSUB = 16
NW = NSC * NSUB
CH = 80
K = 5
BUF = 3
NG = 25
NCH = NG * K
EPW = NCH * CH
NP = 10240
STRIPE = NP // NSUB


def _sc_aggregate(feature, ei4, zrows):
    mesh = plsc.VectorSubcoreMesh(core_axis_name="c", subcore_axis_name="s")
    cp = pltpu.CompilerParams()
    if "needs_layout_passes" in pltpu.CompilerParams.__dataclass_fields__:
        cp = dataclasses.replace(cp, needs_layout_passes=False)

    @functools.partial(
        pl.kernel,
        mesh=mesh,
        compiler_params=cp,
        out_type=(jax.ShapeDtypeStruct((NSC, NP, D), jnp.float32),
                  jax.ShapeDtypeStruct((NW, NP), jnp.float32)),
        scratch_types=[
            pltpu.VMEM_SHARED((NP, D), jnp.float32),
            pltpu.VMEM((K, CH), jnp.int32),
            pltpu.VMEM((K, CH), jnp.int32),
            pltpu.VMEM((BUF, CH, D), jnp.float32),
            pltpu.VMEM((NP,), jnp.float32),
        ] + [pltpu.SemaphoreType.DMA] * (2 * BUF),
    )
    def k(f_hbm, ei_hbm, z_hbm, sums_hbm, cnt_hbm, acc_sh, src_v, dst_v,
          rows_v, hist_v, *all_sems):
        sems = all_sems[:BUF]
        ssems = all_sems[BUF:]
        cid = lax.axis_index("c")
        sid = lax.axis_index("s")
        wid = cid * NSUB + sid
        base = wid * EPW

        pltpu.sync_copy(z_hbm, acc_sh.at[pl.ds(sid * STRIPE, STRIPE)])

        @pl.loop(0, NP, step=16)
        def _(i):
            hist_v[pl.ds(i, 16)] = jnp.zeros((16,), jnp.float32)

        plsc.subcore_barrier()
        ones16 = jnp.ones((16,), jnp.float32)

        @pl.loop(0, NG)
        def _(g):
            pltpu.sync_copy(ei_hbm.at[0, wid, g], src_v)
            pltpu.sync_copy(ei_hbm.at[1, wid, g], dst_v)
            for p in range(min(BUF - 1, K)):
                pltpu.async_copy(f_hbm.at[src_v.at[p]], rows_v.at[p], sems[p])
            for c in range(K):
                b = c % BUF
                for j in range(CH // 16):
                    iv = dst_v[c, pl.ds(j * 16, 16)]
                    plsc.addupdate_scatter(hist_v, [iv], ones16)
                pltpu.make_async_copy(f_hbm.at[src_v.at[c]],
                                      rows_v.at[b], sems[b]).wait()
                pltpu.async_copy(rows_v.at[b], acc_sh.at[dst_v.at[c]],
                                 ssems[b], add=True)
                ahead = c + BUF - 1
                if ahead < K:
                    ba = ahead % BUF
                    if c >= 1:
                        pltpu.make_async_copy(
                            rows_v.at[ba], acc_sh.at[dst_v.at[c - 1]],
                            ssems[ba]).wait()
                    pltpu.async_copy(f_hbm.at[src_v.at[ahead]],
                                     rows_v.at[ba], sems[ba])
            for x in range(max(0, K - BUF), K):
                pltpu.make_async_copy(rows_v.at[x % BUF],
                                      acc_sh.at[dst_v.at[x]],
                                      ssems[x % BUF]).wait()

        pltpu.sync_copy(hist_v, cnt_hbm.at[wid])
        plsc.subcore_barrier()
        pltpu.sync_copy(acc_sh.at[pl.ds(sid * STRIPE, STRIPE)],
                        sums_hbm.at[cid, pl.ds(sid * STRIPE, STRIPE)])

    return k(feature, ei4, zrows)


def _tc_epilogue(acc, cnt, feature, W, b2):
    def body(acc_ref, c_ref, f_ref, w_ref, b_ref, o_ref):
        sums = acc_ref[0, :N, :] + acc_ref[1, :N, :]
        agg = sums / jnp.maximum(c_ref[...], 1.0)
        h = (jnp.dot(agg, w_ref[:D, :], preferred_element_type=jnp.float32)
             + jnp.dot(f_ref[...], w_ref[D:, :], preferred_element_type=jnp.float32)
             + b_ref[...])
        nrm2 = jnp.sum(h * h, axis=1, keepdims=True)
        o_ref[...] = h * lax.rsqrt(jnp.maximum(nrm2, 1e-24))

    return pl.pallas_call(
        body,
        out_shape=jax.ShapeDtypeStruct((N, D), jnp.float32),
    )(acc, cnt, feature, W, b2)


def kernel(feature, edge_index, W, b):
    zrows = jnp.zeros((STRIPE, D), jnp.float32)
    acc, cparts = _sc_aggregate(
        feature, edge_index.reshape(2, NW, NG, K, CH), zrows)
    cnt = cparts.sum(axis=0)[:N, None]
    return _tc_epilogue(acc, cnt, feature, W, b.reshape(1, D))


# speedup vs baseline: 4.7309x; 1.2832x over previous
import dataclasses
import functools

import jax
import jax.numpy as jnp
from jax import lax
from jax.experimental import pallas as pl
from jax.experimental.pallas import tpu as pltpu
from jax.experimental.pallas import tpu_sc as plsc

N = 10000
E = 320000
D = 128
NSC = 2
NSUB = 16
NW = NSC * NSUB
CH = 80
K = 25
BUF = 3
NG = 5
NCH = NG * K
EPW = NCH * CH
NP = 10240
STRIPE = NP // NSUB


def _sc_aggregate(feature, ei4, zrows):
    mesh = plsc.VectorSubcoreMesh(core_axis_name="c", subcore_axis_name="s")
    cp = pltpu.CompilerParams()
    if "needs_layout_passes" in pltpu.CompilerParams.__dataclass_fields__:
        cp = dataclasses.replace(cp, needs_layout_passes=False)

    @functools.partial(
        pl.kernel,
        mesh=mesh,
        compiler_params=cp,
        out_type=(jax.ShapeDtypeStruct((NSC, NP, D), jnp.float32),
                  jax.ShapeDtypeStruct((NW, NP), jnp.float32)),
        scratch_types=[
            pltpu.VMEM_SHARED((NP, D), jnp.float32),
            pltpu.VMEM((K, CH), jnp.int32),
            pltpu.VMEM((K, CH), jnp.int32),
            pltpu.VMEM((BUF, CH, D), jnp.float32),
            pltpu.VMEM((NP,), jnp.float32),
        ] + [pltpu.SemaphoreType.DMA] * (2 * BUF),
    )
    def k(f_hbm, ei_hbm, z_hbm, sums_hbm, cnt_hbm, acc_sh, src_v, dst_v,
          rows_v, hist_v, *all_sems):
        sems = all_sems[:BUF]
        ssems = all_sems[BUF:]
        cid = lax.axis_index("c")
        sid = lax.axis_index("s")
        wid = cid * NSUB + sid
        base = wid * EPW

        pltpu.sync_copy(z_hbm, acc_sh.at[pl.ds(sid * STRIPE, STRIPE)])

        @pl.loop(0, NP, step=16)
        def _(i):
            hist_v[pl.ds(i, 16)] = jnp.zeros((16,), jnp.float32)

        plsc.subcore_barrier()
        ones16 = jnp.ones((16,), jnp.float32)

        @pl.loop(0, NG)
        def _(g):
            pltpu.sync_copy(ei_hbm.at[0, wid, g], src_v)
            pltpu.sync_copy(ei_hbm.at[1, wid, g], dst_v)
            for p in range(min(BUF - 1, K)):
                pltpu.async_copy(f_hbm.at[src_v.at[p]], rows_v.at[p], sems[p])
            for c in range(K):
                b = c % BUF
                for j in range(CH // 16):
                    iv = dst_v[c, pl.ds(j * 16, 16)]
                    plsc.addupdate_scatter(hist_v, [iv], ones16)
                pltpu.make_async_copy(f_hbm.at[src_v.at[c]],
                                      rows_v.at[b], sems[b]).wait()
                pltpu.async_copy(rows_v.at[b], acc_sh.at[dst_v.at[c]],
                                 ssems[b], add=True)
                ahead = c + BUF - 1
                if ahead < K:
                    ba = ahead % BUF
                    if c >= 1:
                        pltpu.make_async_copy(
                            rows_v.at[ba], acc_sh.at[dst_v.at[c - 1]],
                            ssems[ba]).wait()
                    pltpu.async_copy(f_hbm.at[src_v.at[ahead]],
                                     rows_v.at[ba], sems[ba])
            for x in range(max(0, K - BUF), K):
                pltpu.make_async_copy(rows_v.at[x % BUF],
                                      acc_sh.at[dst_v.at[x]],
                                      ssems[x % BUF]).wait()

        pltpu.sync_copy(hist_v, cnt_hbm.at[wid])
        plsc.subcore_barrier()
        pltpu.sync_copy(acc_sh.at[pl.ds(sid * STRIPE, STRIPE)],
                        sums_hbm.at[cid, pl.ds(sid * STRIPE, STRIPE)])

    return k(feature, ei4, zrows)


def _tc_epilogue(acc, cnt, feature, W, b2):
    def body(acc_ref, c_ref, f_ref, w_ref, b_ref, o_ref):
        sums = acc_ref[0, :N, :] + acc_ref[1, :N, :]
        agg = sums / jnp.maximum(c_ref[...], 1.0)
        h = (jnp.dot(agg, w_ref[:D, :], preferred_element_type=jnp.float32)
             + jnp.dot(f_ref[...], w_ref[D:, :], preferred_element_type=jnp.float32)
             + b_ref[...])
        nrm2 = jnp.sum(h * h, axis=1, keepdims=True)
        o_ref[...] = h * lax.rsqrt(jnp.maximum(nrm2, 1e-24))

    return pl.pallas_call(
        body,
        out_shape=jax.ShapeDtypeStruct((N, D), jnp.float32),
    )(acc, cnt, feature, W, b2)


def kernel(feature, edge_index, W, b):
    zrows = jnp.zeros((STRIPE, D), jnp.float32)
    acc, cparts = _sc_aggregate(
        feature, edge_index.reshape(2, NW, NG, K, CH), zrows)
    cnt = cparts.sum(axis=0)[:N, None]
    return _tc_epilogue(acc, cnt, feature, W, b.reshape(1, D))
